# baseline (speedup 1.0000x reference)
import functools

import jax
import jax.numpy as jnp
from jax import lax
from jax.experimental import pallas as pl
from jax.experimental.pallas import tpu as pltpu
from jax.experimental.pallas import tpu_sc as plsc

B = 16384
D = 32
NUM_TABLES = 4
V_SMALL = 1000
V_STORE = 100000
CH = 1024
NCH = B // CH
L = 16
UNROLL = 8

_info = plsc.get_sparse_core_info()
NC = _info.num_cores
NS = _info.num_subcores
NW = NC * NS

STREAM_CHUNKS = [(0, c) for c in range(NCH)] + [(1, c) for c in range(NCH)]
VECTOR_CHUNKS = ([(2, c) for c in range(NCH)]
                 + [(3, c) for c in range(NCH)])
NSB = 3
NVB = 2


@functools.partial(
    pl.kernel,
    out_type=jax.ShapeDtypeStruct((NUM_TABLES, D, B), jnp.float32),
    mesh=plsc.VectorSubcoreMesh(core_axis_name="c", subcore_axis_name="s"),
    compiler_params=pltpu.CompilerParams(use_tc_tiling_on_sc=True,
                                         needs_layout_passes=False),
    scratch_types=(
        [pltpu.VMEM((V_STORE,), jnp.float32)]
        + [pltpu.VMEM((V_SMALL,), jnp.float32)]
        + [pltpu.VMEM((V_SMALL,), jnp.float32)] * 2
        + [pltpu.VMEM_SHARED((NS * V_SMALL,), jnp.float32)] * 2
        + [pltpu.VMEM_SHARED((NUM_TABLES * B,), jnp.int32)]
        + [pltpu.VMEM((CH,), jnp.int32)] * (NVB + NSB)
        + [pltpu.VMEM((CH,), jnp.float32)] * (NVB + NSB)
        + [pltpu.SemaphoreType.DMA] * 20
    ),
)
def _encode(cat_h, col_h, fab_h, store_h,
            cat_t, col_t, fab_t, store_t,
            out_h,
            store_row, fab_row, b0, b1, sh0, sh1, idx_sh,
            iv0, iv1, is0, is1, is2, ov0, ov1, os0, os1, os2,
            *sems):
    sid = lax.axis_index("s")
    k = sid * NC + lax.axis_index("c")
    idx_srcs = (cat_h, col_h, fab_h, store_h)
    row_sems = sems[0:4]
    v2s_sems = sems[4:6]
    vidx_sems = sems[6:8]
    sidx_sems = sems[8:11]
    vout_sems = sems[11:13]
    sout_sems = sems[13:16]
    g_sems = sems[16:19]
    stg_sem = sems[19]

    vec_rows = {1: b1, 2: fab_row, 3: store_row}
    sh_rows = {0: sh0, 1: sh1}
    iv_bufs = (iv0, iv1)
    is_bufs = (is0, is1, is2)
    ov_bufs = (ov0, ov1)
    os_bufs = (os0, os1, os2)

    row_cp = {
        0: pltpu.async_copy(cat_t.at[k], b0, row_sems[0]),
        1: pltpu.async_copy(col_t.at[k], b1, row_sems[1]),
        2: pltpu.async_copy(fab_t.at[k], fab_row, row_sems[2]),
        3: pltpu.async_copy(store_t.at[k], store_row, row_sems[3]),
    }

    @pl.when(sid == 0)
    def _stage_indices():
        st_cp = [pltpu.async_copy(idx_srcs[t],
                                  idx_sh.at[pl.ds(t * B, B)],
                                  stg_sem)
                 for t in range(NUM_TABLES)]
        for cp in st_cp:
            cp.wait()
    plsc.subcore_barrier()

    v2s_cp = {}
    for t, bb in ((0, b0), (1, b1)):
        row_cp[t].wait()
        v2s_cp[t] = pltpu.async_copy(
            bb, sh_rows[t].at[pl.ds(sid * V_SMALL, V_SMALL)], v2s_sems[t])

    def idx_slice(t, c):
        return idx_sh.at[pl.ds((t * B) + c * CH, CH)]

    sidx_cp = {}
    vidx_cp = {}
    g_cp = {}
    out_cp = {}
    v2s_waited = set()
    row_waited = {0, 1}

    sidx_cp[0] = pltpu.async_copy(idx_slice(*STREAM_CHUNKS[0]),
                                  is_bufs[0], sidx_sems[0])
    vidx_cp[0] = pltpu.async_copy(idx_slice(*VECTOR_CHUNKS[0]),
                                  iv_bufs[0], vidx_sems[0])

    for p in range(len(VECTOR_CHUNKS)):
        if p < len(STREAM_CHUNKS):
            ts, cs = STREAM_CHUNKS[p]
            if p + 1 < len(STREAM_CHUNKS):
                nb = (p + 1) % NSB
                sidx_cp[p + 1] = pltpu.async_copy(
                    idx_slice(*STREAM_CHUNKS[p + 1]), is_bufs[nb],
                    sidx_sems[nb])
            if ts not in v2s_waited:
                v2s_cp[ts].wait()
                v2s_waited.add(ts)
            if p - NSB >= 0 and f"s{p - NSB}" in out_cp:
                out_cp.pop(f"s{p - NSB}").wait()
            sidx_cp.pop(p).wait()
            g_cp[p] = pltpu.async_copy(
                sh_rows[ts].at[pl.ds(sid * V_SMALL, V_SMALL)]
                .at[is_bufs[p % NSB]],
                os_bufs[p % NSB], g_sems[p % NSB])
        if p - 1 in g_cp:
            g_cp.pop(p - 1).wait()
            tso, cso = STREAM_CHUNKS[p - 1]
            out_cp[f"s{p - 1}"] = pltpu.async_copy(
                os_bufs[(p - 1) % NSB],
                out_h.at[tso, k, pl.ds(cso * CH, CH)],
                sout_sems[(p - 1) % NSB])

        tv, cv = VECTOR_CHUNKS[p]
        if p + 1 < len(VECTOR_CHUNKS):
            nb = (p + 1) % NVB
            vidx_cp[p + 1] = pltpu.async_copy(
                idx_slice(*VECTOR_CHUNKS[p + 1]), iv_bufs[nb], vidx_sems[nb])
        if tv not in row_waited:
            row_cp[tv].wait()
            row_waited.add(tv)
        if p - NVB >= 0 and f"v{p - NVB}" in out_cp:
            out_cp.pop(f"v{p - NVB}").wait()
        vidx_cp.pop(p).wait()
        ib = iv_bufs[p % NVB]
        ob = ov_bufs[p % NVB]
        row = vec_rows[tv]

        def body(i, _):
            base = i * (L * UNROLL)
            for u in range(UNROLL):
                ivec = ib[pl.ds(base + u * L, L)]
                ob[pl.ds(base + u * L, L)] = plsc.load_gather(row, [ivec])
            return 0

        lax.fori_loop(0, CH // (L * UNROLL), body, 0)
        out_cp[f"v{p}"] = pltpu.async_copy(
            ob, out_h.at[tv, k, pl.ds(cv * CH, CH)], vout_sems[p % NVB])

    last = len(STREAM_CHUNKS) - 1
    if last in g_cp:
        g_cp.pop(last).wait()
        tso, cso = STREAM_CHUNKS[last]
        out_cp[f"s{last}"] = pltpu.async_copy(
            os_bufs[last % NSB], out_h.at[tso, k, pl.ds(cso * CH, CH)],
            sout_sems[last % NSB])
    for key in sorted(out_cp):
        out_cp.pop(key).wait()


def kernel(cat, col, fab, store, cat_table, col_table, fab_table, store_table):
    out_phys = _encode(cat, col, fab, store,
                       cat_table.T, col_table.T, fab_table.T, store_table.T)
    return jnp.transpose(out_phys, (2, 0, 1))


# speedup vs baseline: 1.2725x; 1.2725x over previous
import functools

import jax
import jax.numpy as jnp
from jax import lax
from jax.experimental import pallas as pl
from jax.experimental.pallas import tpu as pltpu
from jax.experimental.pallas import tpu_sc as plsc

B = 16384
D = 32
NUM_TABLES = 4
V_SMALL = 1000
V_STORE = 100000
CH = 2048
NCH = B // CH
L = 16
UNROLL = 8

_info = plsc.get_sparse_core_info()
NC = _info.num_cores
NS = _info.num_subcores
NW = NC * NS

STREAM_CHUNKS = [(0, c) for c in range(NCH)] + [(1, c) for c in range(NCH)]
VECTOR_CHUNKS = ([(2, c) for c in range(NCH)]
                 + [(3, c) for c in range(NCH)])
NSB = 3
NVB = 2


@functools.partial(
    pl.kernel,
    out_type=jax.ShapeDtypeStruct((NUM_TABLES, D, B), jnp.float32),
    mesh=plsc.VectorSubcoreMesh(core_axis_name="c", subcore_axis_name="s"),
    compiler_params=pltpu.CompilerParams(use_tc_tiling_on_sc=True,
                                         needs_layout_passes=False),
    scratch_types=(
        [pltpu.VMEM((V_STORE,), jnp.float32)]
        + [pltpu.VMEM((V_SMALL,), jnp.float32)]
        + [pltpu.VMEM((V_SMALL,), jnp.float32)] * 2
        + [pltpu.VMEM_SHARED((NS * V_SMALL,), jnp.float32)] * 2
        + [pltpu.VMEM_SHARED((NUM_TABLES * B,), jnp.int32)]
        + [pltpu.VMEM((CH,), jnp.int32)] * (NVB + NSB)
        + [pltpu.VMEM((CH,), jnp.float32)] * (NVB + NSB)
        + [pltpu.SemaphoreType.DMA] * 20
    ),
)
def _encode(cat_h, col_h, fab_h, store_h,
            cat_t, col_t, fab_t, store_t,
            out_h,
            store_row, fab_row, b0, b1, sh0, sh1, idx_sh,
            iv0, iv1, is0, is1, is2, ov0, ov1, os0, os1, os2,
            *sems):
    sid = lax.axis_index("s")
    k = sid * NC + lax.axis_index("c")
    idx_srcs = (cat_h, col_h, fab_h, store_h)
    row_sems = sems[0:4]
    v2s_sems = sems[4:6]
    vidx_sems = sems[6:8]
    sidx_sems = sems[8:11]
    vout_sems = sems[11:13]
    sout_sems = sems[13:16]
    g_sems = sems[16:19]
    stg_sem = sems[19]

    vec_rows = {1: b1, 2: fab_row, 3: store_row}
    sh_rows = {0: sh0, 1: sh1}
    iv_bufs = (iv0, iv1)
    is_bufs = (is0, is1, is2)
    ov_bufs = (ov0, ov1)
    os_bufs = (os0, os1, os2)

    row_cp = {
        0: pltpu.async_copy(cat_t.at[k], b0, row_sems[0]),
        1: pltpu.async_copy(col_t.at[k], b1, row_sems[1]),
        2: pltpu.async_copy(fab_t.at[k], fab_row, row_sems[2]),
        3: pltpu.async_copy(store_t.at[k], store_row, row_sems[3]),
    }

    @pl.when(sid == 0)
    def _stage_indices():
        st_cp = [pltpu.async_copy(idx_srcs[t],
                                  idx_sh.at[pl.ds(t * B, B)],
                                  stg_sem)
                 for t in range(NUM_TABLES)]
        for cp in st_cp:
            cp.wait()
    plsc.subcore_barrier()

    v2s_cp = {}
    for t, bb in ((0, b0), (1, b1)):
        row_cp[t].wait()
        v2s_cp[t] = pltpu.async_copy(
            bb, sh_rows[t].at[pl.ds(sid * V_SMALL, V_SMALL)], v2s_sems[t])

    def idx_slice(t, c):
        return idx_sh.at[pl.ds((t * B) + c * CH, CH)]

    sidx_cp = {}
    vidx_cp = {}
    g_cp = {}
    out_cp = {}
    v2s_waited = set()
    row_waited = {0, 1}

    sidx_cp[0] = pltpu.async_copy(idx_slice(*STREAM_CHUNKS[0]),
                                  is_bufs[0], sidx_sems[0])
    vidx_cp[0] = pltpu.async_copy(idx_slice(*VECTOR_CHUNKS[0]),
                                  iv_bufs[0], vidx_sems[0])

    for p in range(len(VECTOR_CHUNKS)):
        if p < len(STREAM_CHUNKS):
            ts, cs = STREAM_CHUNKS[p]
            if p + 1 < len(STREAM_CHUNKS):
                nb = (p + 1) % NSB
                sidx_cp[p + 1] = pltpu.async_copy(
                    idx_slice(*STREAM_CHUNKS[p + 1]), is_bufs[nb],
                    sidx_sems[nb])
            if ts not in v2s_waited:
                v2s_cp[ts].wait()
                v2s_waited.add(ts)
            if p - NSB >= 0 and f"s{p - NSB}" in out_cp:
                out_cp.pop(f"s{p - NSB}").wait()
            sidx_cp.pop(p).wait()
            g_cp[p] = pltpu.async_copy(
                sh_rows[ts].at[pl.ds(sid * V_SMALL, V_SMALL)]
                .at[is_bufs[p % NSB]],
                os_bufs[p % NSB], g_sems[p % NSB])
        if p - 1 in g_cp:
            g_cp.pop(p - 1).wait()
            tso, cso = STREAM_CHUNKS[p - 1]
            out_cp[f"s{p - 1}"] = pltpu.async_copy(
                os_bufs[(p - 1) % NSB],
                out_h.at[tso, k, pl.ds(cso * CH, CH)],
                sout_sems[(p - 1) % NSB])

        tv, cv = VECTOR_CHUNKS[p]
        if p + 1 < len(VECTOR_CHUNKS):
            nb = (p + 1) % NVB
            vidx_cp[p + 1] = pltpu.async_copy(
                idx_slice(*VECTOR_CHUNKS[p + 1]), iv_bufs[nb], vidx_sems[nb])
        if tv not in row_waited:
            row_cp[tv].wait()
            row_waited.add(tv)
        if p - NVB >= 0 and f"v{p - NVB}" in out_cp:
            out_cp.pop(f"v{p - NVB}").wait()
        vidx_cp.pop(p).wait()
        ib = iv_bufs[p % NVB]
        ob = ov_bufs[p % NVB]
        row = vec_rows[tv]

        def body(i, _):
            base = i * (L * UNROLL)
            for u in range(UNROLL):
                ivec = ib[pl.ds(base + u * L, L)]
                ob[pl.ds(base + u * L, L)] = plsc.load_gather(row, [ivec])
            return 0

        lax.fori_loop(0, CH // (L * UNROLL), body, 0)
        out_cp[f"v{p}"] = pltpu.async_copy(
            ob, out_h.at[tv, k, pl.ds(cv * CH, CH)], vout_sems[p % NVB])

    last = len(STREAM_CHUNKS) - 1
    if last in g_cp:
        g_cp.pop(last).wait()
        tso, cso = STREAM_CHUNKS[last]
        out_cp[f"s{last}"] = pltpu.async_copy(
            os_bufs[last % NSB], out_h.at[tso, k, pl.ds(cso * CH, CH)],
            sout_sems[last % NSB])
    for key in sorted(out_cp):
        out_cp.pop(key).wait()


def kernel(cat, col, fab, store, cat_table, col_table, fab_table, store_table):
    out_phys = _encode(cat, col, fab, store,
                       cat_table.T, col_table.T, fab_table.T, store_table.T)
    return jnp.transpose(out_phys, (2, 0, 1))
